# baseline (speedup 1.0000x reference)
import functools

import jax
import jax.numpy as jnp
from jax import lax
from jax.experimental import pallas as pl
from jax.experimental.pallas import tpu as pltpu
from jax.experimental.pallas import tpu_sc as plsc

N_NODES = 10000
N_EDGES = 320000
N_REL = 8
NR = N_NODES * N_REL

NC = 2
NS = 16
LANES = 16

CHUNK = 128
N_PAD = 10240
NR_PAD = N_PAD * N_REL
E_PAD = 327680
NCH = E_PAD // CHUNK
BLK = 8
NBLK = NCH // BLK
CPT = NCH // (NC * NS)
ROWS_PER_TILE = N_PAD // NS
CNT_SLICE = NR_PAD // NS

_SC_MESH = plsc.VectorSubcoreMesh(core_axis_name="c", subcore_axis_name="s")


def _prep_body(src_hbm, dst_hbm, typ_hbm, gkey_hbm, skey_hbm, inv_hbm,
               src_b, dst_b, typ_b, gkey_b, skey_b, ones_c, zbuf, cnt_sh,
               sem):
    cid = lax.axis_index("c")
    sid = lax.axis_index("s")

    def zero_body(i, _):
        zbuf[pl.ds(i * LANES, LANES)] = jnp.zeros((LANES,), jnp.float32)
        return 0
    lax.fori_loop(0, CNT_SLICE // LANES, zero_body, 0)
    pltpu.sync_copy(zbuf, cnt_sh.at[pl.ds(sid * CNT_SLICE, CNT_SLICE)])
    for g in range(CHUNK // LANES):
        ones_c[pl.ds(g * LANES, LANES)] = jnp.ones((LANES,), jnp.float32)
    plsc.subcore_barrier()

    nt = NBLK // NS

    @pl.when(cid == 0)
    def _():
        def count_body(t, _):
            j = (sid + NS * t) * BLK
            pltpu.sync_copy(dst_hbm.at[pl.ds(j, BLK)], dst_b)
            pltpu.sync_copy(typ_hbm.at[pl.ds(j, BLK)], typ_b)
            for u in range(BLK):
                for g in range(CHUNK // LANES):
                    sl = pl.ds(g * LANES, LANES)
                    skey_b[u, sl] = dst_b[u, sl] * N_REL + typ_b[u, sl]
            for u in range(BLK):
                pltpu.async_copy(ones_c, cnt_sh.at[skey_b.at[u]], sem, add=True)
            for u in range(BLK):
                pltpu.make_async_copy(ones_c, cnt_sh.at[skey_b.at[u]], sem).wait()
            return 0
        lax.fori_loop(0, nt, count_body, 0)

    @pl.when(cid == 1)
    def _():
        def key_body(t, _):
            j = (sid + NS * t) * BLK
            pltpu.sync_copy(src_hbm.at[pl.ds(j, BLK)], src_b)
            pltpu.sync_copy(dst_hbm.at[pl.ds(j, BLK)], dst_b)
            pltpu.sync_copy(typ_hbm.at[pl.ds(j, BLK)], typ_b)
            for u in range(BLK):
                for g in range(CHUNK // LANES):
                    sl = pl.ds(g * LANES, LANES)
                    gkey_b[u, sl] = src_b[u, sl] * N_REL + typ_b[u, sl]
                    skey_b[u, sl] = dst_b[u, sl] * N_REL + typ_b[u, sl]
            pltpu.sync_copy(gkey_b, gkey_hbm.at[pl.ds(j, BLK)])
            pltpu.sync_copy(skey_b, skey_hbm.at[pl.ds(j, BLK)])
            return 0
        lax.fori_loop(0, nt, key_body, 0)

    plsc.subcore_barrier()

    @pl.when(cid == 0)
    def _():
        pltpu.sync_copy(cnt_sh.at[pl.ds(sid * CNT_SLICE, CNT_SLICE)], zbuf)
        def inv_body(i, _):
            sl = pl.ds(i * LANES, LANES)
            zbuf[sl] = 1.0 / jnp.maximum(zbuf[sl], 1.0)
            return 0
        lax.fori_loop(0, CNT_SLICE // LANES, inv_body, 0)
        pltpu.sync_copy(zbuf, inv_hbm.at[pl.ds(sid * CNT_SLICE, CNT_SLICE)])


_prep = pl.kernel(
    _prep_body,
    out_type=(
        jax.ShapeDtypeStruct((NCH, CHUNK), jnp.int32),
        jax.ShapeDtypeStruct((NCH, CHUNK), jnp.int32),
        jax.ShapeDtypeStruct((NR_PAD,), jnp.float32),
    ),
    mesh=_SC_MESH,
    scratch_types=[
        pltpu.VMEM((BLK, CHUNK), jnp.int32),
        pltpu.VMEM((BLK, CHUNK), jnp.int32),
        pltpu.VMEM((BLK, CHUNK), jnp.int32),
        pltpu.VMEM((BLK, CHUNK), jnp.int32),
        pltpu.VMEM((BLK, CHUNK), jnp.int32),
        pltpu.VMEM((CHUNK,), jnp.float32),
        pltpu.VMEM((CNT_SLICE,), jnp.float32),
        pltpu.VMEM_SHARED((NR_PAD,), jnp.float32),
        pltpu.SemaphoreType.DMA,
    ],
)


NBLOCKS = CPT // BLK


def _scatter_body(active, y_hbm, gkey_hbm, skey_hbm, inv_hbm,
                  part_hbm, gk0, gk1, sk0, sk1, db0, db1, w0, w1,
                  rows0, rows1, acc, g0, g1, ws0, ws1, ss0, ss1, k0, k1):
    cid = lax.axis_index("c")
    sid = lax.axis_index("s")
    wid = sid * NC + cid
    start = wid * CPT

    gk = (gk0, gk1)
    sk = (sk0, sk1)
    db = (db0, db1)
    rows = (rows0, rows1)
    wv_ = (w0, w1)
    gsem = (g0, g1)
    wsem = (ws0, ws1)
    ssem = (ss0, ss1)
    ksem = (k0, k1)

    def derive_db(ks):
        for u in range(BLK):
            for g in range(CHUNK // LANES):
                sl = pl.ds(g * LANES, LANES)
                db[ks][u, sl] = lax.shift_right_logical(sk[ks][u, sl], 3)

    def zero_body(i, _):
        for g in range(128 // LANES):
            rows0[i, pl.ds(g * LANES, LANES)] = jnp.zeros((LANES,), jnp.float32)
        return 0
    lax.fori_loop(0, CHUNK, zero_body, 0)
    base = sid * ROWS_PER_TILE
    for k in range(ROWS_PER_TILE // CHUNK):
        pltpu.sync_copy(rows0, acc.at[pl.ds(base + k * CHUNK, CHUNK)])
    plsc.subcore_barrier()

    pltpu.sync_copy(gkey_hbm.at[pl.ds(start, BLK)], gk0)
    pltpu.sync_copy(skey_hbm.at[pl.ds(start, BLK)], sk0)
    derive_db(0)
    pltpu.async_copy(y_hbm.at[gk0.at[0]], rows0, g0)
    pltpu.async_copy(inv_hbm.at[sk0.at[0]], w0, ws0)

    def block_body(u2, _):
        for ub in range(2):
            u = 2 * u2 + ub
            ks = ub
            nk = 1 - ub

            @pl.when(u + 1 < NBLOCKS)
            def _():
                j = start + (u + 1) * BLK
                pltpu.async_copy(gkey_hbm.at[pl.ds(j, BLK)], gk[nk], ksem[nk])
                pltpu.async_copy(skey_hbm.at[pl.ds(j, BLK)], sk[nk], ksem[nk])

            for c in range(BLK):
                t = u * BLK + c
                b = c % 2
                nb = 1 - b

                if c < BLK - 1:
                    @pl.when(t >= 1)
                    def _():
                        pltpu.make_async_copy(
                            rows[nb], acc.at[db[ks].at[c]], ssem[nb]).wait()
                    pltpu.async_copy(y_hbm.at[gk[ks].at[c + 1]],
                                     rows[nb], gsem[nb])
                    pltpu.async_copy(inv_hbm.at[sk[ks].at[c + 1]],
                                     wv_[nb], wsem[nb])
                else:
                    @pl.when(u + 1 < NBLOCKS)
                    def _():
                        pltpu.make_async_copy(
                            rows[nb], acc.at[db[ks].at[c]], ssem[nb]).wait()
                        pltpu.make_async_copy(
                            gkey_hbm.at[pl.ds(start, BLK)], gk[nk],
                            ksem[nk]).wait()
                        pltpu.make_async_copy(
                            skey_hbm.at[pl.ds(start, BLK)], sk[nk],
                            ksem[nk]).wait()
                        derive_db(nk)
                        pltpu.async_copy(y_hbm.at[gk[nk].at[0]],
                                         rows[nb], gsem[nb])
                        pltpu.async_copy(inv_hbm.at[sk[nk].at[0]],
                                         wv_[nb], wsem[nb])

                pltpu.make_async_copy(
                    y_hbm.at[gk[ks].at[c]], rows[b], gsem[b]).wait()
                pltpu.make_async_copy(
                    inv_hbm.at[sk[ks].at[c]], wv_[b], wsem[b]).wait()

                def scale_body(g, _):
                    wvec = wv_[b][pl.ds(g * LANES, LANES)]
                    for k in range(LANES):
                        wi = wvec[k]
                        i = g * LANES + k
                        for q in range(active // LANES):
                            sl = pl.ds(q * LANES, LANES)
                            rows[b][i, sl] = rows[b][i, sl] * wi
                    return 0
                lax.fori_loop(0, CHUNK // LANES, scale_body, 0)

                pltpu.async_copy(rows[b], acc.at[db[ks].at[c]],
                                 ssem[b], add=True)
        return 0
    lax.fori_loop(0, NBLOCKS // 2, block_body, 0)

    pltpu.make_async_copy(rows0, acc.at[db0.at[0]], ss0).wait()
    pltpu.make_async_copy(rows1, acc.at[db0.at[0]], ss1).wait()

    plsc.subcore_barrier()
    pltpu.sync_copy(acc.at[pl.ds(base, ROWS_PER_TILE)],
                    part_hbm.at[cid, pl.ds(base, ROWS_PER_TILE)])


def _make_scatter(active):
    return pl.kernel(
        functools.partial(_scatter_body, active),
        out_type=jax.ShapeDtypeStruct((NC, N_PAD, 128), jnp.float32),
        mesh=_SC_MESH,
        scratch_types=[
            pltpu.VMEM((BLK, CHUNK), jnp.int32),
            pltpu.VMEM((BLK, CHUNK), jnp.int32),
            pltpu.VMEM((BLK, CHUNK), jnp.int32),
            pltpu.VMEM((BLK, CHUNK), jnp.int32),
            pltpu.VMEM((BLK, CHUNK), jnp.int32),
            pltpu.VMEM((BLK, CHUNK), jnp.int32),
            pltpu.VMEM((CHUNK,), jnp.float32),
            pltpu.VMEM((CHUNK,), jnp.float32),
            pltpu.VMEM((CHUNK, 128), jnp.float32),
            pltpu.VMEM((CHUNK, 128), jnp.float32),
            pltpu.VMEM_SHARED((N_PAD, 128), jnp.float32),
            pltpu.SemaphoreType.DMA,
            pltpu.SemaphoreType.DMA,
            pltpu.SemaphoreType.DMA,
            pltpu.SemaphoreType.DMA,
            pltpu.SemaphoreType.DMA,
            pltpu.SemaphoreType.DMA,
            pltpu.SemaphoreType.DMA,
            pltpu.SemaphoreType.DMA,
        ],
    )


_scatter128 = _make_scatter(128)
_scatter64 = _make_scatter(64)


_BN = 2000


def _transform_body(x_ref, w_ref, root_ref, b_ref, y_ref, self_ref):
    x = x_ref[...]
    xh = x.astype(jnp.bfloat16)
    self_ref[...] = (
        jnp.dot(x, root_ref[...], preferred_element_type=jnp.float32)
        + b_ref[...]
    )
    for r in range(N_REL):
        y_ref[:, r, :] = jnp.dot(xh, w_ref[r], preferred_element_type=jnp.float32)


def _transform(x, w, root, b):
    n, d_in = x.shape
    d_y = w.shape[2]
    d_self = root.shape[1]
    y, self_out = pl.pallas_call(
        _transform_body,
        grid=(n // _BN,),
        in_specs=[
            pl.BlockSpec((_BN, d_in), lambda i: (i, 0)),
            pl.BlockSpec((N_REL, d_in, d_y), lambda i: (0, 0, 0)),
            pl.BlockSpec((d_in, d_self), lambda i: (0, 0)),
            pl.BlockSpec((1, d_self), lambda i: (0, 0)),
        ],
        out_specs=[
            pl.BlockSpec((_BN, N_REL, d_y), lambda i: (i, 0, 0)),
            pl.BlockSpec((_BN, d_self), lambda i: (i, 0)),
        ],
        out_shape=[
            jax.ShapeDtypeStruct((n, N_REL, d_y), jnp.float32),
            jax.ShapeDtypeStruct((n, d_self), jnp.float32),
        ],
    )(x, w, root, b.reshape(1, d_self))
    return y.reshape(n * N_REL, d_y), self_out


def _elu(s):
    return jnp.where(s > 0, s, jnp.exp(jnp.minimum(s, 0.0)) - 1.0)


def _ctransform_body(s_ref, p_ref, w_ref, root_ref, b_ref,
                     h_ref, y_ref, self_ref):
    h = _elu(s_ref[...] + p_ref[0] + p_ref[1])
    h_ref[...] = h
    hh = h.astype(jnp.bfloat16)
    self_ref[...] = (
        jnp.dot(h, root_ref[...], preferred_element_type=jnp.float32)
        + b_ref[...]
    )
    for r in range(N_REL):
        y_ref[:, r, :] = jnp.dot(hh, w_ref[r], preferred_element_type=jnp.float32)


def _ctransform(self_prev, parts, w, root, b):
    n = self_prev.shape[0]
    act = self_prev.shape[1]
    d_y = w.shape[2]
    d_self = root.shape[1]
    h, y, self_out = pl.pallas_call(
        _ctransform_body,
        grid=(n // _BN,),
        in_specs=[
            pl.BlockSpec((_BN, act), lambda i: (i, 0)),
            pl.BlockSpec((2, _BN, act), lambda i: (0, i, 0)),
            pl.BlockSpec((N_REL, act, d_y), lambda i: (0, 0, 0)),
            pl.BlockSpec((act, d_self), lambda i: (0, 0)),
            pl.BlockSpec((1, d_self), lambda i: (0, 0)),
        ],
        out_specs=[
            pl.BlockSpec((_BN, act), lambda i: (i, 0)),
            pl.BlockSpec((_BN, N_REL, d_y), lambda i: (i, 0, 0)),
            pl.BlockSpec((_BN, d_self), lambda i: (i, 0)),
        ],
        out_shape=[
            jax.ShapeDtypeStruct((n, act), jnp.float32),
            jax.ShapeDtypeStruct((n, N_REL, d_y), jnp.float32),
            jax.ShapeDtypeStruct((n, d_self), jnp.float32),
        ],
    )(self_prev, parts, w, root, b.reshape(1, d_self))
    return h, y.reshape(n * N_REL, d_y), self_out


def _combine_body(s_ref, p_ref, o_ref):
    o_ref[...] = _elu(s_ref[...] + p_ref[0] + p_ref[1])


def _combine(self_out, parts):
    n, d = self_out.shape
    return pl.pallas_call(
        _combine_body,
        grid=(n // _BN,),
        in_specs=[
            pl.BlockSpec((_BN, d), lambda i: (i, 0)),
            pl.BlockSpec((2, _BN, d), lambda i: (0, i, 0)),
        ],
        out_specs=pl.BlockSpec((_BN, d), lambda i: (i, 0)),
        out_shape=jax.ShapeDtypeStruct((n, d), jnp.float32),
    )(self_out, parts)


def kernel(x, edge_index, edge_type, W1, root1, b1, W2, root2, b2,
           W3, root3, b3):
    npad = E_PAD - N_EDGES
    pad_ids = jax.lax.iota(jnp.int32, npad)
    src = jnp.concatenate(
        [edge_index[0], pad_ids % N_NODES]).reshape(NCH, CHUNK)
    dst = jnp.concatenate(
        [edge_index[1], N_NODES + pad_ids % (N_PAD - N_NODES)]
    ).reshape(NCH, CHUNK)
    typ = jnp.concatenate(
        [edge_type, jnp.zeros((npad,), jnp.int32)]).reshape(NCH, CHUNK)

    gkey, skey, inv = _prep(src, dst, typ)

    pad = ((0, 0), (0, 0), (0, 64))
    y1, self1 = _transform(x, W1.astype(jnp.bfloat16), root1, b1)
    parts1 = _scatter128(y1, gkey, skey, inv)
    h1, y2, self2 = _ctransform(self1, parts1[:, :N_NODES],
                                jnp.pad(W2, pad).astype(jnp.bfloat16),
                                root2, b2)
    parts2 = _scatter64(y2, gkey, skey, inv)
    h2, y3, self3 = _ctransform(self2, parts2[:, :N_NODES, :64],
                                jnp.pad(W3, pad).astype(jnp.bfloat16),
                                root3, b3)
    parts3 = _scatter64(y3, gkey, skey, inv)
    h3 = _combine(self3, parts3[:, :N_NODES, :64])
    return jnp.concatenate([h1, h2, h3], axis=1)


# speedup vs baseline: 1.0031x; 1.0031x over previous
import functools

import jax
import jax.numpy as jnp
from jax import lax
from jax.experimental import pallas as pl
from jax.experimental.pallas import tpu as pltpu
from jax.experimental.pallas import tpu_sc as plsc

N_NODES = 10000
N_EDGES = 320000
N_REL = 8
NR = N_NODES * N_REL

NC = 2
NS = 16
LANES = 16

CHUNK = 128
N_PAD = 10240
NR_PAD = N_PAD * N_REL
E_PAD = 327680
NCH = E_PAD // CHUNK
BLK = 8
NBLK = NCH // BLK
CPT = NCH // (NC * NS)
ROWS_PER_TILE = N_PAD // NS
CNT_SLICE = NR_PAD // NS

_SC_MESH = plsc.VectorSubcoreMesh(core_axis_name="c", subcore_axis_name="s")


def _prep_body(src_hbm, dst_hbm, typ_hbm, gkey_hbm, skey_hbm, inv_hbm,
               src_b, dst_b, typ_b, gkey_b, skey_b, ones_c, zbuf, cnt_sh,
               sem):
    cid = lax.axis_index("c")
    sid = lax.axis_index("s")

    def zero_body(i, _):
        zbuf[pl.ds(i * LANES, LANES)] = jnp.zeros((LANES,), jnp.float32)
        return 0
    lax.fori_loop(0, CNT_SLICE // LANES, zero_body, 0)
    pltpu.sync_copy(zbuf, cnt_sh.at[pl.ds(sid * CNT_SLICE, CNT_SLICE)])
    for g in range(CHUNK // LANES):
        ones_c[pl.ds(g * LANES, LANES)] = jnp.ones((LANES,), jnp.float32)
    plsc.subcore_barrier()

    nt = NBLK // NS

    @pl.when(cid == 0)
    def _():
        def count_body(t, _):
            j = (sid + NS * t) * BLK
            pltpu.sync_copy(dst_hbm.at[pl.ds(j, BLK)], dst_b)
            pltpu.sync_copy(typ_hbm.at[pl.ds(j, BLK)], typ_b)
            for u in range(BLK):
                for g in range(CHUNK // LANES):
                    sl = pl.ds(g * LANES, LANES)
                    skey_b[u, sl] = dst_b[u, sl] * N_REL + typ_b[u, sl]
            for u in range(BLK):
                pltpu.async_copy(ones_c, cnt_sh.at[skey_b.at[u]], sem, add=True)
            for u in range(BLK):
                pltpu.make_async_copy(ones_c, cnt_sh.at[skey_b.at[u]], sem).wait()
            return 0
        lax.fori_loop(0, nt, count_body, 0)

    @pl.when(cid == 1)
    def _():
        def key_body(t, _):
            j = (sid + NS * t) * BLK
            pltpu.sync_copy(src_hbm.at[pl.ds(j, BLK)], src_b)
            pltpu.sync_copy(dst_hbm.at[pl.ds(j, BLK)], dst_b)
            pltpu.sync_copy(typ_hbm.at[pl.ds(j, BLK)], typ_b)
            for u in range(BLK):
                for g in range(CHUNK // LANES):
                    sl = pl.ds(g * LANES, LANES)
                    gkey_b[u, sl] = src_b[u, sl] * N_REL + typ_b[u, sl]
                    skey_b[u, sl] = dst_b[u, sl] * N_REL + typ_b[u, sl]
            pltpu.sync_copy(gkey_b, gkey_hbm.at[pl.ds(j, BLK)])
            pltpu.sync_copy(skey_b, skey_hbm.at[pl.ds(j, BLK)])
            return 0
        lax.fori_loop(0, nt, key_body, 0)

    plsc.subcore_barrier()

    @pl.when(cid == 0)
    def _():
        pltpu.sync_copy(cnt_sh.at[pl.ds(sid * CNT_SLICE, CNT_SLICE)], zbuf)
        def inv_body(i, _):
            sl = pl.ds(i * LANES, LANES)
            zbuf[sl] = 1.0 / jnp.maximum(zbuf[sl], 1.0)
            return 0
        lax.fori_loop(0, CNT_SLICE // LANES, inv_body, 0)
        pltpu.sync_copy(zbuf, inv_hbm.at[pl.ds(sid * CNT_SLICE, CNT_SLICE)])


_prep = pl.kernel(
    _prep_body,
    out_type=(
        jax.ShapeDtypeStruct((NCH, CHUNK), jnp.int32),
        jax.ShapeDtypeStruct((NCH, CHUNK), jnp.int32),
        jax.ShapeDtypeStruct((NR_PAD,), jnp.float32),
    ),
    mesh=_SC_MESH,
    scratch_types=[
        pltpu.VMEM((BLK, CHUNK), jnp.int32),
        pltpu.VMEM((BLK, CHUNK), jnp.int32),
        pltpu.VMEM((BLK, CHUNK), jnp.int32),
        pltpu.VMEM((BLK, CHUNK), jnp.int32),
        pltpu.VMEM((BLK, CHUNK), jnp.int32),
        pltpu.VMEM((CHUNK,), jnp.float32),
        pltpu.VMEM((CNT_SLICE,), jnp.float32),
        pltpu.VMEM_SHARED((NR_PAD,), jnp.float32),
        pltpu.SemaphoreType.DMA,
    ],
)


NBLOCKS = CPT // BLK


def _scatter_body(active, y_hbm, gkey_hbm, skey_hbm, inv_hbm,
                  part_hbm, gk0, gk1, sk0, sk1, db0, db1, w0, w1,
                  rows0, rows1, acc, g0, g1, ws0, ws1, ss0, ss1, k0, k1):
    cid = lax.axis_index("c")
    sid = lax.axis_index("s")
    wid = sid * NC + cid
    start = wid * CPT

    gk = (gk0, gk1)
    sk = (sk0, sk1)
    db = (db0, db1)
    rows = (rows0, rows1)
    wv_ = (w0, w1)
    gsem = (g0, g1)
    wsem = (ws0, ws1)
    ssem = (ss0, ss1)
    ksem = (k0, k1)

    def derive_db(ks):
        for u in range(BLK):
            for g in range(CHUNK // LANES):
                sl = pl.ds(g * LANES, LANES)
                db[ks][u, sl] = lax.shift_right_logical(sk[ks][u, sl], 3)

    def zero_body(i, _):
        for g in range(128 // LANES):
            rows0[i, pl.ds(g * LANES, LANES)] = jnp.zeros((LANES,), jnp.float32)
        return 0
    lax.fori_loop(0, CHUNK, zero_body, 0)
    base = sid * ROWS_PER_TILE
    for k in range(ROWS_PER_TILE // CHUNK):
        pltpu.sync_copy(rows0, acc.at[pl.ds(base + k * CHUNK, CHUNK)])
    plsc.subcore_barrier()

    pltpu.sync_copy(gkey_hbm.at[pl.ds(start, BLK)], gk0)
    pltpu.sync_copy(skey_hbm.at[pl.ds(start, BLK)], sk0)
    derive_db(0)
    pltpu.async_copy(y_hbm.at[gk0.at[0]], rows0, g0)
    pltpu.async_copy(inv_hbm.at[sk0.at[0]], w0, ws0)

    def block_body(u2, _):
        for ub in range(2):
            u = 2 * u2 + ub
            ks = ub
            nk = 1 - ub

            @pl.when(u + 1 < NBLOCKS)
            def _():
                j = start + (u + 1) * BLK
                pltpu.async_copy(gkey_hbm.at[pl.ds(j, BLK)], gk[nk], ksem[nk])
                pltpu.async_copy(skey_hbm.at[pl.ds(j, BLK)], sk[nk], ksem[nk])

            for c in range(BLK):
                t = u * BLK + c
                b = c % 2
                nb = 1 - b

                if c < BLK - 1:
                    @pl.when(t >= 1)
                    def _():
                        pltpu.make_async_copy(
                            rows[nb], acc.at[db[ks].at[c]], ssem[nb]).wait()
                    pltpu.async_copy(y_hbm.at[gk[ks].at[c + 1]],
                                     rows[nb], gsem[nb])
                    pltpu.async_copy(inv_hbm.at[sk[ks].at[c + 1]],
                                     wv_[nb], wsem[nb])
                else:
                    @pl.when(u + 1 < NBLOCKS)
                    def _():
                        pltpu.make_async_copy(
                            rows[nb], acc.at[db[ks].at[c]], ssem[nb]).wait()
                        pltpu.make_async_copy(
                            gkey_hbm.at[pl.ds(start, BLK)], gk[nk],
                            ksem[nk]).wait()
                        pltpu.make_async_copy(
                            skey_hbm.at[pl.ds(start, BLK)], sk[nk],
                            ksem[nk]).wait()
                        derive_db(nk)
                        pltpu.async_copy(y_hbm.at[gk[nk].at[0]],
                                         rows[nb], gsem[nb])
                        pltpu.async_copy(inv_hbm.at[sk[nk].at[0]],
                                         wv_[nb], wsem[nb])

                pltpu.make_async_copy(
                    y_hbm.at[gk[ks].at[c]], rows[b], gsem[b]).wait()
                pltpu.make_async_copy(
                    inv_hbm.at[sk[ks].at[c]], wv_[b], wsem[b]).wait()

                def scale_body(g, _):
                    wvec = wv_[b][pl.ds(g * LANES, LANES)]
                    for k in range(LANES):
                        wi = wvec[k]
                        i = g * LANES + k
                        for q in range(active // LANES):
                            sl = pl.ds(q * LANES, LANES)
                            rows[b][i, sl] = rows[b][i, sl] * wi
                    return 0
                lax.fori_loop(0, CHUNK // LANES, scale_body, 0)

                pltpu.async_copy(rows[b], acc.at[db[ks].at[c]],
                                 ssem[b], add=True)
        return 0
    lax.fori_loop(0, NBLOCKS // 2, block_body, 0)

    pltpu.make_async_copy(rows0, acc.at[db0.at[0]], ss0).wait()
    pltpu.make_async_copy(rows1, acc.at[db0.at[0]], ss1).wait()

    plsc.subcore_barrier()
    pltpu.sync_copy(acc.at[pl.ds(base, ROWS_PER_TILE)],
                    part_hbm.at[cid, pl.ds(base, ROWS_PER_TILE)])


def _make_scatter(active):
    return pl.kernel(
        functools.partial(_scatter_body, active),
        out_type=jax.ShapeDtypeStruct((NC, N_PAD, 128), jnp.float32),
        mesh=_SC_MESH,
        scratch_types=[
            pltpu.VMEM((BLK, CHUNK), jnp.int32),
            pltpu.VMEM((BLK, CHUNK), jnp.int32),
            pltpu.VMEM((BLK, CHUNK), jnp.int32),
            pltpu.VMEM((BLK, CHUNK), jnp.int32),
            pltpu.VMEM((BLK, CHUNK), jnp.int32),
            pltpu.VMEM((BLK, CHUNK), jnp.int32),
            pltpu.VMEM((CHUNK,), jnp.float32),
            pltpu.VMEM((CHUNK,), jnp.float32),
            pltpu.VMEM((CHUNK, 128), jnp.float32),
            pltpu.VMEM((CHUNK, 128), jnp.float32),
            pltpu.VMEM_SHARED((N_PAD, 128), jnp.float32),
            pltpu.SemaphoreType.DMA,
            pltpu.SemaphoreType.DMA,
            pltpu.SemaphoreType.DMA,
            pltpu.SemaphoreType.DMA,
            pltpu.SemaphoreType.DMA,
            pltpu.SemaphoreType.DMA,
            pltpu.SemaphoreType.DMA,
            pltpu.SemaphoreType.DMA,
        ],
    )


_scatter128 = _make_scatter(128)
_scatter64 = _make_scatter(64)


_BN = 2000


def _transform_body(x_ref, w_ref, root_ref, b_ref, y_ref, self_ref):
    x = x_ref[...]
    self_ref[...] = (
        jnp.dot(x, root_ref[...], preferred_element_type=jnp.float32)
        + b_ref[...]
    )
    for r in range(N_REL):
        y_ref[:, r, :] = jnp.dot(x, w_ref[r], preferred_element_type=jnp.float32)


def _transform(x, w, root, b):
    n, d_in = x.shape
    d_y = w.shape[2]
    d_self = root.shape[1]
    y, self_out = pl.pallas_call(
        _transform_body,
        grid=(n // _BN,),
        in_specs=[
            pl.BlockSpec((_BN, d_in), lambda i: (i, 0)),
            pl.BlockSpec((N_REL, d_in, d_y), lambda i: (0, 0, 0)),
            pl.BlockSpec((d_in, d_self), lambda i: (0, 0)),
            pl.BlockSpec((1, d_self), lambda i: (0, 0)),
        ],
        out_specs=[
            pl.BlockSpec((_BN, N_REL, d_y), lambda i: (i, 0, 0)),
            pl.BlockSpec((_BN, d_self), lambda i: (i, 0)),
        ],
        out_shape=[
            jax.ShapeDtypeStruct((n, N_REL, d_y), jnp.float32),
            jax.ShapeDtypeStruct((n, d_self), jnp.float32),
        ],
    )(x, w, root, b.reshape(1, d_self))
    return y.reshape(n * N_REL, d_y), self_out


def _elu(s):
    return jnp.where(s > 0, s, jnp.exp(jnp.minimum(s, 0.0)) - 1.0)


def _ctransform_body(s_ref, p_ref, w_ref, root_ref, b_ref,
                     h_ref, y_ref, self_ref):
    h = _elu(s_ref[...] + p_ref[0] + p_ref[1])
    h_ref[...] = h
    self_ref[...] = (
        jnp.dot(h, root_ref[...], preferred_element_type=jnp.float32)
        + b_ref[...]
    )
    for r in range(N_REL):
        y_ref[:, r, :] = jnp.dot(h, w_ref[r], preferred_element_type=jnp.float32)


def _ctransform(self_prev, parts, w, root, b):
    n = self_prev.shape[0]
    act = self_prev.shape[1]
    d_y = w.shape[2]
    d_self = root.shape[1]
    h, y, self_out = pl.pallas_call(
        _ctransform_body,
        grid=(n // _BN,),
        in_specs=[
            pl.BlockSpec((_BN, act), lambda i: (i, 0)),
            pl.BlockSpec((2, _BN, act), lambda i: (0, i, 0)),
            pl.BlockSpec((N_REL, act, d_y), lambda i: (0, 0, 0)),
            pl.BlockSpec((act, d_self), lambda i: (0, 0)),
            pl.BlockSpec((1, d_self), lambda i: (0, 0)),
        ],
        out_specs=[
            pl.BlockSpec((_BN, act), lambda i: (i, 0)),
            pl.BlockSpec((_BN, N_REL, d_y), lambda i: (i, 0, 0)),
            pl.BlockSpec((_BN, d_self), lambda i: (i, 0)),
        ],
        out_shape=[
            jax.ShapeDtypeStruct((n, act), jnp.float32),
            jax.ShapeDtypeStruct((n, N_REL, d_y), jnp.float32),
            jax.ShapeDtypeStruct((n, d_self), jnp.float32),
        ],
    )(self_prev, parts, w, root, b.reshape(1, d_self))
    return h, y.reshape(n * N_REL, d_y), self_out


def _combine_body(s_ref, p_ref, o_ref):
    o_ref[...] = _elu(s_ref[...] + p_ref[0] + p_ref[1])


def _combine(self_out, parts):
    n, d = self_out.shape
    return pl.pallas_call(
        _combine_body,
        grid=(n // _BN,),
        in_specs=[
            pl.BlockSpec((_BN, d), lambda i: (i, 0)),
            pl.BlockSpec((2, _BN, d), lambda i: (0, i, 0)),
        ],
        out_specs=pl.BlockSpec((_BN, d), lambda i: (i, 0)),
        out_shape=jax.ShapeDtypeStruct((n, d), jnp.float32),
    )(self_out, parts)


def kernel(x, edge_index, edge_type, W1, root1, b1, W2, root2, b2,
           W3, root3, b3):
    npad = E_PAD - N_EDGES
    pad_ids = jax.lax.iota(jnp.int32, npad)
    src = jnp.concatenate(
        [edge_index[0], pad_ids % N_NODES]).reshape(NCH, CHUNK)
    dst = jnp.concatenate(
        [edge_index[1], N_NODES + pad_ids % (N_PAD - N_NODES)]
    ).reshape(NCH, CHUNK)
    typ = jnp.concatenate(
        [edge_type, jnp.zeros((npad,), jnp.int32)]).reshape(NCH, CHUNK)

    gkey, skey, inv = _prep(src, dst, typ)

    pad = ((0, 0), (0, 0), (0, 64))
    y1, self1 = _transform(x, W1, root1, b1)
    parts1 = _scatter128(y1, gkey, skey, inv)
    h1, y2, self2 = _ctransform(self1, parts1[:, :N_NODES],
                                jnp.pad(W2, pad), root2, b2)
    parts2 = _scatter64(y2, gkey, skey, inv)
    h2, y3, self3 = _ctransform(self2, parts2[:, :N_NODES, :64],
                                jnp.pad(W3, pad), root3, b3)
    parts3 = _scatter64(y3, gkey, skey, inv)
    h3 = _combine(self3, parts3[:, :N_NODES, :64])
    return jnp.concatenate([h1, h2, h3], axis=1)
